# Initial kernel scaffold; baseline (speedup 1.0000x reference)
import functools

import jax
import jax.numpy as jnp
from jax import lax
from jax.experimental import pallas as pl
from jax.experimental.pallas import tpu as pltpu
from jax.experimental.pallas import tpu_sc as plsc

N = 50000
E = 800000
NG = 512
H = 64
HH = 32

NC = 2
NS = 16
SLAB = 3136
NPAD = NS * SLAB
BN = SLAB
GRID = NPAD // BN

F32 = jnp.float32

_MESH = plsc.VectorSubcoreMesh(core_axis_name="c", subcore_axis_name="s")


EC = E // (NC * NS)
DC = 1000
EPS = E // NS
AC = 2000


def _fill_f32(ref, n, val):
    def body(i, _):
        ref[pl.ds(i * 16, 16)] = jnp.full((16,), val, F32)
        return 0
    lax.fori_loop(0, n // 16, body, 0)


def _deg_body(dst_hbm, out0, out1, dst_v, ones_v, zrow_v, acc, sem):
    c = lax.axis_index("c")
    s = lax.axis_index("s")
    _fill_f32(zrow_v, SLAB, 0.0)
    _fill_f32(ones_v, DC, 1.0)
    pltpu.sync_copy(zrow_v, acc.at[pl.ds(s * SLAB, SLAB)])
    plsc.subcore_barrier()
    wid = s * NC + c
    def chunk(k, _):
        base = wid * EC + k * DC
        pltpu.sync_copy(dst_hbm.at[pl.ds(base, DC)], dst_v)
        pltpu.async_copy(ones_v, acc.at[dst_v], sem, add=True).wait()
        return 0
    lax.fori_loop(0, EC // DC, chunk, 0)
    plsc.subcore_barrier()
    sl = pl.ds(s * SLAB, SLAB)
    @pl.when(c == 0)
    def _():
        pltpu.sync_copy(acc.at[sl], out0.at[sl])
    @pl.when(c == 1)
    def _():
        pltpu.sync_copy(acc.at[sl], out1.at[sl])


_deg_call = pl.kernel(
    _deg_body,
    out_type=(jax.ShapeDtypeStruct((NPAD,), F32),
              jax.ShapeDtypeStruct((NPAD,), F32)),
    mesh=_MESH,
    scratch_types=[
        pltpu.VMEM((DC,), jnp.int32),
        pltpu.VMEM((DC,), F32),
        pltpu.VMEM((SLAB,), F32),
        pltpu.VMEM_SHARED((NPAD,), F32),
        pltpu.SemaphoreType.DMA,
    ],
)


def _agg8_body(tab_hbm, src_hbm, dst_hbm, out0, out1,
               src_v, dst_v, rows_v, zrows_v, acc, sem):
    c = lax.axis_index("c")
    s = lax.axis_index("s")
    _fill_f32(zrows_v.reshape(SLAB * 8), SLAB * 8, 0.0)
    pltpu.sync_copy(zrows_v, acc.at[pl.ds(s * SLAB, SLAB)])
    plsc.subcore_barrier()
    wid = s * NC + c
    def chunk(k, _):
        base = wid * EC + k * DC
        pltpu.sync_copy(src_hbm.at[pl.ds(base, DC)], src_v)
        pltpu.sync_copy(dst_hbm.at[pl.ds(base, DC)], dst_v)
        pltpu.async_copy(tab_hbm.at[src_v], rows_v, sem).wait()
        pltpu.async_copy(rows_v, acc.at[dst_v], sem, add=True).wait()
        return 0
    lax.fori_loop(0, EC // DC, chunk, 0)
    plsc.subcore_barrier()
    sl = pl.ds(s * SLAB, SLAB)
    @pl.when(c == 0)
    def _():
        pltpu.sync_copy(acc.at[sl], out0.at[sl])
    @pl.when(c == 1)
    def _():
        pltpu.sync_copy(acc.at[sl], out1.at[sl])


_agg8_call = pl.kernel(
    _agg8_body,
    out_type=(jax.ShapeDtypeStruct((NPAD, 8), F32),
              jax.ShapeDtypeStruct((NPAD, 8), F32)),
    mesh=_MESH,
    scratch_types=[
        pltpu.VMEM((DC,), jnp.int32),
        pltpu.VMEM((DC,), jnp.int32),
        pltpu.VMEM((DC, 8), F32),
        pltpu.VMEM((SLAB, 8), F32),
        pltpu.VMEM_SHARED((NPAD, 8), F32),
        pltpu.SemaphoreType.DMA,
    ],
)


def _agg32_body(taba_hbm, tabb_hbm, srcd_hbm, dstd_hbm, outa, outb,
                src_v, dst_v, rows_v, zrows_v, acc, sem):
    c = lax.axis_index("c")
    s = lax.axis_index("s")

    def half(tab_hbm, out):
        _fill_f32(zrows_v.reshape(784 * 32), 784 * 32, 0.0)
        for j in range(SLAB // 784):
            pltpu.sync_copy(zrows_v, acc.at[pl.ds(s * SLAB + j * 784, 784)])
        plsc.subcore_barrier()
        def chunk(k, _):
            base = s * EPS + k * AC
            pltpu.sync_copy(srcd_hbm.at[pl.ds(base, AC)], src_v)
            pltpu.sync_copy(dstd_hbm.at[pl.ds(base, AC)], dst_v)
            pltpu.async_copy(tab_hbm.at[src_v], rows_v, sem).wait()
            pltpu.async_copy(rows_v, acc.at[dst_v], sem, add=True).wait()
            return 0
        lax.fori_loop(0, EPS // AC, chunk, 0)
        plsc.subcore_barrier()
        sl = pl.ds(s * SLAB, SLAB)
        pltpu.sync_copy(acc.at[sl], out.at[sl])

    @pl.when(c == 0)
    def _():
        half(taba_hbm, outa)
    @pl.when(c == 1)
    def _():
        half(tabb_hbm, outb)


_agg32_call = pl.kernel(
    _agg32_body,
    out_type=(jax.ShapeDtypeStruct((NPAD, HH), F32),
              jax.ShapeDtypeStruct((NPAD, HH), F32)),
    mesh=_MESH,
    scratch_types=[
        pltpu.VMEM((AC,), jnp.int32),
        pltpu.VMEM((AC,), jnp.int32),
        pltpu.VMEM((AC, HH), F32),
        pltpu.VMEM((784, HH), F32),
        pltpu.VMEM_SHARED((NPAD, HH), F32),
        pltpu.SemaphoreType.DMA,
    ],
)


def _row_spec(w):
    return pl.BlockSpec((BN, w), lambda i: (i, 0))


def _full_spec(a, b):
    return pl.BlockSpec((a, b), lambda i: (0, 0))


def _prep_body(x_ref, da_ref, db_ref, dinv_ref, gx_ref):
    deg = da_ref[...] + db_ref[...] + 1.0
    dinv = lax.rsqrt(deg)
    dinv_ref[...] = dinv
    xs = x_ref[...] * dinv
    gx_ref[...] = jnp.concatenate([xs, jnp.zeros((BN, 6), F32)], axis=1)


def _prep_call(xp, da, db):
    return pl.pallas_call(
        _prep_body,
        grid=(GRID,),
        in_specs=[_row_spec(2), _row_spec(1), _row_spec(1)],
        out_specs=(_row_spec(1), _row_spec(8)),
        out_shape=(jax.ShapeDtypeStruct((NPAD, 1), F32),
                   jax.ShapeDtypeStruct((NPAD, 8), F32)),
    )(xp, da, db)


def _l1_body(t0a, t0b, gx, dinv, w1, b1, w2, ga_ref, gb_ref):
    q = t0a[...] + t0b[...] + gx[...]
    xin = q[:, 0:2] * dinv[...]
    y1 = jnp.maximum(
        jnp.dot(xin, w1[...], preferred_element_type=F32) + b1[...], 0.0)
    h2 = jnp.dot(y1, w2[...], preferred_element_type=F32)
    g1 = h2 * dinv[...]
    ga_ref[...] = g1[:, 0:HH]
    gb_ref[...] = g1[:, HH:H]


def _l1_call(t0a, t0b, gx, dinv, w1, b1, w2):
    return pl.pallas_call(
        _l1_body,
        grid=(GRID,),
        in_specs=[_row_spec(8), _row_spec(8), _row_spec(8), _row_spec(1),
                  _full_spec(2, H), _full_spec(1, H), _full_spec(H, H)],
        out_specs=(_row_spec(HH), _row_spec(HH)),
        out_shape=(jax.ShapeDtypeStruct((NPAD, HH), F32),
                   jax.ShapeDtypeStruct((NPAD, HH), F32)),
    )(t0a, t0b, gx, dinv, w1, b1, w2)


def _l2_body(ta, tb, ga, gb, dinv, b2, w3, oa_ref, ob_ref):
    d = dinv[...]
    u0 = jnp.maximum((ta[...] + ga[...]) * d + b2[...][:, 0:HH], 0.0)
    u1 = jnp.maximum((tb[...] + gb[...]) * d + b2[...][:, HH:H], 0.0)
    h3 = (jnp.dot(u0, w3[...][0:HH, :], preferred_element_type=F32)
          + jnp.dot(u1, w3[...][HH:H, :], preferred_element_type=F32))
    g2 = h3 * d
    oa_ref[...] = g2[:, 0:HH]
    ob_ref[...] = g2[:, HH:H]


def _l2_call(ta, tb, ga, gb, dinv, b2, w3):
    return pl.pallas_call(
        _l2_body,
        grid=(GRID,),
        in_specs=[_row_spec(HH)] * 4 + [_row_spec(1),
                  _full_spec(1, H), _full_spec(H, H)],
        out_specs=(_row_spec(HH), _row_spec(HH)),
        out_shape=(jax.ShapeDtypeStruct((NPAD, HH), F32),
                   jax.ShapeDtypeStruct((NPAD, HH), F32)),
    )(ta, tb, ga, gb, dinv, b2, w3)


def _pool_body(ta, tb, ga, gb, dinv, b3, batch, s0_ref, s1_ref, cnt_ref):
    i = pl.program_id(0)
    d = dinv[...]
    v0 = jnp.maximum((ta[...] + ga[...]) * d + b3[...][:, 0:HH], 0.0)
    v1 = jnp.maximum((tb[...] + gb[...]) * d + b3[...][:, HH:H], 0.0)
    ids = batch[...]
    seg = lax.broadcasted_iota(jnp.int32, (BN, NG), 1)
    p = (ids == seg).astype(F32)
    dn = (((0,), (0,)), ((), ()))
    s0 = lax.dot_general(p, v0, dn, preferred_element_type=F32)
    s1 = lax.dot_general(p, v1, dn, preferred_element_type=F32)
    cnt = lax.dot_general(p, jnp.ones((BN, 1), F32), dn,
                          preferred_element_type=F32)

    @pl.when(i == 0)
    def _():
        s0_ref[...] = jnp.zeros_like(s0_ref)
        s1_ref[...] = jnp.zeros_like(s1_ref)
        cnt_ref[...] = jnp.zeros_like(cnt_ref)

    s0_ref[...] += s0
    s1_ref[...] += s1
    cnt_ref[...] += cnt


def _pool_call(ta, tb, ga, gb, dinv, b3, batchp):
    return pl.pallas_call(
        _pool_body,
        grid=(GRID,),
        in_specs=[_row_spec(HH)] * 4 + [_row_spec(1), _full_spec(1, H),
                  pl.BlockSpec((BN, 1), lambda i: (i, 0))],
        out_specs=(_full_spec(NG, HH), _full_spec(NG, HH), _full_spec(NG, 1)),
        out_shape=(jax.ShapeDtypeStruct((NG, HH), F32),
                   jax.ShapeDtypeStruct((NG, HH), F32),
                   jax.ShapeDtypeStruct((NG, 1), F32)),
    )(ta, tb, ga, gb, dinv, b3, batchp)


def _head_body(s0, s1, cnt, wf1, bf1, wf2, bf2, out_ref):
    denom = jnp.maximum(cnt[...], 1.0)
    p0 = s0[...] / denom
    p1 = s1[...] / denom
    h = jnp.maximum(
        jnp.dot(p0, wf1[...][0:HH, :], preferred_element_type=F32)
        + jnp.dot(p1, wf1[...][HH:H, :], preferred_element_type=F32)
        + bf1[...], 0.0)
    logits = jnp.dot(h, wf2[...], preferred_element_type=F32) + bf2[...]
    m = jnp.max(logits, axis=1, keepdims=True)
    e = jnp.exp(logits - m)
    lse = jnp.log(jnp.sum(e, axis=1, keepdims=True)) + m
    out_ref[...] = logits - lse


def _head_call(s0, s1, cnt, wf1, bf1, wf2, bf2):
    return pl.pallas_call(
        _head_body,
        out_shape=jax.ShapeDtypeStruct((NG, 10), F32),
    )(s0, s1, cnt, wf1, bf1, wf2, bf2)


def kernel(x, edge_index, batch, W1, b1, W2, b2, W3, b3, Wf1, bf1, Wf2, bf2):
    src = edge_index[0]
    dst = edge_index[1]
    xp = jnp.pad(x, ((0, NPAD - N), (0, 0)))
    batchp = jnp.pad(batch, (0, NPAD - N), constant_values=-1).reshape(NPAD, 1)

    dega, degb = _deg_call(dst)
    dinv, gx = _prep_call(xp, dega.reshape(NPAD, 1), degb.reshape(NPAD, 1))

    t0a, t0b = _agg8_call(gx, src, dst)
    g1a, g1b = _l1_call(t0a, t0b, gx, dinv, W1, b1.reshape(1, H), W2)

    t1a, t1b = _agg32_call(g1a, g1b, src, dst)
    g2a, g2b = _l2_call(t1a, t1b, g1a, g1b, dinv, b2.reshape(1, H), W3)

    t2a, t2b = _agg32_call(g2a, g2b, src, dst)
    s0, s1, cnt = _pool_call(t2a, t2b, g2a, g2b, dinv, b3.reshape(1, H),
                             batchp)

    return _head_call(s0, s1, cnt, Wf1, bf1.reshape(1, HH), Wf2,
                      bf2.reshape(1, 10))


# speedup vs baseline: 23.0821x; 23.0821x over previous
import functools

import jax
import jax.numpy as jnp
from jax import lax
from jax.experimental import pallas as pl
from jax.experimental.pallas import tpu as pltpu
from jax.experimental.pallas import tpu_sc as plsc

N = 50000
E = 800000
NG = 512
H = 64
HH = 32

NC = 2
NS = 16
SLAB = 3136
NPAD = NS * SLAB
BN = SLAB
GRID = NPAD // BN

F32 = jnp.float32

@functools.cache
def _mesh():
    return plsc.VectorSubcoreMesh(core_axis_name="c", subcore_axis_name="s",
                                  num_cores=NC, num_subcores=NS)


EC = E // (NC * NS)
DC = 1000
EPS = E // NS
AC = 400
STG = 392


def _fill_f32(ref, n, val):
    def body(i, _):
        ref[pl.ds(i * 16, 16)] = jnp.full((16,), val, F32)
        return 0
    lax.fori_loop(0, n // 16, body, 0)


def _deg_body(dst_hbm, out0, out1, dst_v, ones_v, zrow_v, acc, sem):
    c = lax.axis_index("c")
    s = lax.axis_index("s")
    _fill_f32(zrow_v, SLAB, 0.0)
    _fill_f32(ones_v, DC, 1.0)
    pltpu.sync_copy(zrow_v, acc.at[pl.ds(s * SLAB, SLAB)])
    plsc.subcore_barrier()
    wid = s * NC + c
    def chunk(k, _):
        base = wid * EC + k * DC
        pltpu.sync_copy(dst_hbm.at[pl.ds(base, DC)], dst_v)
        pltpu.async_copy(ones_v, acc.at[dst_v], sem, add=True).wait()
        return 0
    lax.fori_loop(0, EC // DC, chunk, 0)
    plsc.subcore_barrier()
    sl = pl.ds(s * SLAB, SLAB)
    pltpu.sync_copy(acc.at[sl], zrow_v)
    @pl.when(c == 0)
    def _():
        pltpu.sync_copy(zrow_v, out0.at[sl])
    @pl.when(c == 1)
    def _():
        pltpu.sync_copy(zrow_v, out1.at[sl])


@functools.cache
def _deg_call():
    return pl.kernel(
    _deg_body,
    out_type=(jax.ShapeDtypeStruct((NPAD,), F32),
              jax.ShapeDtypeStruct((NPAD,), F32)),
    mesh=_mesh(),
    scratch_types=[
        pltpu.VMEM((DC,), jnp.int32),
        pltpu.VMEM((DC,), F32),
        pltpu.VMEM((SLAB,), F32),
        pltpu.VMEM_SHARED((NPAD,), F32),
        pltpu.SemaphoreType.DMA,
    ],
    compiler_params=pltpu.CompilerParams(use_tc_tiling_on_sc=False),
    )


def _agg8_body(tab_hbm, src_hbm, dst_hbm, z_hbm, out0, out1,
               src_v, dst_v, rows_v, slab_v, acc, sem):
    c = lax.axis_index("c")
    s = lax.axis_index("s")
    for j in range(SLAB // 784):
        zsl = pl.ds(s * SLAB + j * 784, 784)
        pltpu.sync_copy(z_hbm.at[zsl], slab_v)
        pltpu.sync_copy(slab_v, acc.at[zsl])
    plsc.subcore_barrier()
    wid = s * NC + c
    def chunk(k, _):
        base = wid * EC + k * DC
        pltpu.sync_copy(src_hbm.at[pl.ds(base, DC)], src_v)
        pltpu.sync_copy(dst_hbm.at[pl.ds(base, DC)], dst_v)
        pltpu.async_copy(tab_hbm.at[src_v], rows_v, sem).wait()
        pltpu.async_copy(rows_v, acc.at[dst_v], sem, add=True).wait()
        return 0
    lax.fori_loop(0, EC // DC, chunk, 0)
    plsc.subcore_barrier()
    for j in range(SLAB // 784):
        sl = pl.ds(s * SLAB + j * 784, 784)
        pltpu.sync_copy(acc.at[sl], slab_v)
        @pl.when(c == 0)
        def _():
            pltpu.sync_copy(slab_v, out0.at[sl])
        @pl.when(c == 1)
        def _():
            pltpu.sync_copy(slab_v, out1.at[sl])


@functools.cache
def _agg8_call():
    return pl.kernel(
    _agg8_body,
    out_type=(jax.ShapeDtypeStruct((NPAD, 8), F32),
              jax.ShapeDtypeStruct((NPAD, 8), F32)),
    mesh=_mesh(),
    scratch_types=[
        pltpu.VMEM((DC,), jnp.int32),
        pltpu.VMEM((DC,), jnp.int32),
        pltpu.VMEM((DC, 8), F32),
        pltpu.VMEM((784, 8), F32),
        pltpu.VMEM_SHARED((NPAD, 8), F32),
        pltpu.SemaphoreType.DMA,
    ],
    compiler_params=pltpu.CompilerParams(use_tc_tiling_on_sc=False),
    )


def _fill_rows32(ref, nrows):
    def body(i, _):
        ref[i, pl.ds(0, 16)] = jnp.zeros((16,), F32)
        ref[i, pl.ds(16, 16)] = jnp.zeros((16,), F32)
        return 0
    lax.fori_loop(0, nrows, body, 0)


def _agg32_body(taba_hbm, tabb_hbm, srcd_hbm, dstd_hbm, outa, outb,
                src_v, dst_v, rows_v, acc, sem):
    c = lax.axis_index("c")
    s = lax.axis_index("s")
    stg = rows_v.at[pl.ds(0, STG)]

    def half(tab_hbm, out):
        _fill_rows32(rows_v, STG)
        for j in range(SLAB // STG):
            pltpu.sync_copy(stg, acc.at[pl.ds(s * SLAB + j * STG, STG)])
        plsc.subcore_barrier()
        def chunk(k, _):
            base = s * EPS + k * AC
            pltpu.sync_copy(srcd_hbm.at[pl.ds(base, AC)], src_v)
            pltpu.sync_copy(dstd_hbm.at[pl.ds(base, AC)], dst_v)
            pltpu.async_copy(tab_hbm.at[src_v], rows_v, sem).wait()
            pltpu.async_copy(rows_v, acc.at[dst_v], sem, add=True).wait()
            return 0
        lax.fori_loop(0, EPS // AC, chunk, 0)
        plsc.subcore_barrier()
        for j in range(SLAB // STG):
            sl = pl.ds(s * SLAB + j * STG, STG)
            pltpu.sync_copy(acc.at[sl], stg)
            pltpu.sync_copy(stg, out.at[sl])

    @pl.when(c == 0)
    def _():
        half(taba_hbm, outa)
    @pl.when(c == 1)
    def _():
        half(tabb_hbm, outb)


@functools.cache
def _agg32_call():
    return pl.kernel(
    _agg32_body,
    out_type=(jax.ShapeDtypeStruct((NPAD, HH), F32),
              jax.ShapeDtypeStruct((NPAD, HH), F32)),
    mesh=_mesh(),
    scratch_types=[
        pltpu.VMEM((AC,), jnp.int32),
        pltpu.VMEM((AC,), jnp.int32),
        pltpu.VMEM((AC, HH), F32),
        pltpu.VMEM_SHARED((NPAD, HH), F32),
        pltpu.SemaphoreType.DMA,
    ],
    compiler_params=pltpu.CompilerParams(use_tc_tiling_on_sc=False),
    )


def _row_spec(w):
    return pl.BlockSpec((BN, w), lambda i: (i, 0))


def _full_spec(a, b):
    return pl.BlockSpec((a, b), lambda i: (0, 0))


def _prep_body(x_ref, da_ref, db_ref, dinv_ref, gx_ref):
    deg = da_ref[...] + db_ref[...] + 1.0
    dinv = lax.rsqrt(deg)
    dinv_ref[...] = dinv
    xs = x_ref[...] * dinv
    gx_ref[...] = jnp.concatenate([xs, jnp.zeros((BN, 6), F32)], axis=1)


def _prep_call(xp, da, db):
    return pl.pallas_call(
        _prep_body,
        grid=(GRID,),
        in_specs=[_row_spec(2), _row_spec(1), _row_spec(1)],
        out_specs=(_row_spec(1), _row_spec(8)),
        out_shape=(jax.ShapeDtypeStruct((NPAD, 1), F32),
                   jax.ShapeDtypeStruct((NPAD, 8), F32)),
    )(xp, da, db)


def _l1_body(t0a, t0b, gx, dinv, w1, b1, w2, ga_ref, gb_ref):
    q = t0a[...] + t0b[...] + gx[...]
    xin = q[:, 0:2] * dinv[...]
    y1 = jnp.maximum(
        jnp.dot(xin, w1[...], preferred_element_type=F32) + b1[...], 0.0)
    h2 = jnp.dot(y1, w2[...], preferred_element_type=F32)
    g1 = h2 * dinv[...]
    ga_ref[...] = g1[:, 0:HH]
    gb_ref[...] = g1[:, HH:H]


def _l1_call(t0a, t0b, gx, dinv, w1, b1, w2):
    return pl.pallas_call(
        _l1_body,
        grid=(GRID,),
        in_specs=[_row_spec(8), _row_spec(8), _row_spec(8), _row_spec(1),
                  _full_spec(2, H), _full_spec(1, H), _full_spec(H, H)],
        out_specs=(_row_spec(HH), _row_spec(HH)),
        out_shape=(jax.ShapeDtypeStruct((NPAD, HH), F32),
                   jax.ShapeDtypeStruct((NPAD, HH), F32)),
    )(t0a, t0b, gx, dinv, w1, b1, w2)


def _l2_body(ta, tb, ga, gb, dinv, b2, w3, oa_ref, ob_ref):
    d = dinv[...]
    u0 = jnp.maximum((ta[...] + ga[...]) * d + b2[...][:, 0:HH], 0.0)
    u1 = jnp.maximum((tb[...] + gb[...]) * d + b2[...][:, HH:H], 0.0)
    h3 = (jnp.dot(u0, w3[...][0:HH, :], preferred_element_type=F32)
          + jnp.dot(u1, w3[...][HH:H, :], preferred_element_type=F32))
    g2 = h3 * d
    oa_ref[...] = g2[:, 0:HH]
    ob_ref[...] = g2[:, HH:H]


def _l2_call(ta, tb, ga, gb, dinv, b2, w3):
    return pl.pallas_call(
        _l2_body,
        grid=(GRID,),
        in_specs=[_row_spec(HH)] * 4 + [_row_spec(1),
                  _full_spec(1, H), _full_spec(H, H)],
        out_specs=(_row_spec(HH), _row_spec(HH)),
        out_shape=(jax.ShapeDtypeStruct((NPAD, HH), F32),
                   jax.ShapeDtypeStruct((NPAD, HH), F32)),
    )(ta, tb, ga, gb, dinv, b2, w3)


def _pool_body(ta, tb, ga, gb, dinv, b3, batch, s0_ref, s1_ref, cnt_ref):
    i = pl.program_id(0)
    d = dinv[...]
    v0 = jnp.maximum((ta[...] + ga[...]) * d + b3[...][:, 0:HH], 0.0)
    v1 = jnp.maximum((tb[...] + gb[...]) * d + b3[...][:, HH:H], 0.0)
    ids = batch[...]
    seg = lax.broadcasted_iota(jnp.int32, (BN, NG), 1)
    p = (ids == seg).astype(F32)
    dn = (((0,), (0,)), ((), ()))
    s0 = lax.dot_general(p, v0, dn, preferred_element_type=F32)
    s1 = lax.dot_general(p, v1, dn, preferred_element_type=F32)
    cnt = lax.dot_general(p, jnp.ones((BN, 1), F32), dn,
                          preferred_element_type=F32)

    @pl.when(i == 0)
    def _():
        s0_ref[...] = jnp.zeros_like(s0_ref)
        s1_ref[...] = jnp.zeros_like(s1_ref)
        cnt_ref[...] = jnp.zeros_like(cnt_ref)

    s0_ref[...] += s0
    s1_ref[...] += s1
    cnt_ref[...] += cnt


def _pool_call(ta, tb, ga, gb, dinv, b3, batchp):
    return pl.pallas_call(
        _pool_body,
        grid=(GRID,),
        in_specs=[_row_spec(HH)] * 4 + [_row_spec(1), _full_spec(1, H),
                  pl.BlockSpec((BN, 1), lambda i: (i, 0))],
        out_specs=(_full_spec(NG, HH), _full_spec(NG, HH), _full_spec(NG, 1)),
        out_shape=(jax.ShapeDtypeStruct((NG, HH), F32),
                   jax.ShapeDtypeStruct((NG, HH), F32),
                   jax.ShapeDtypeStruct((NG, 1), F32)),
    )(ta, tb, ga, gb, dinv, b3, batchp)


def _head_body(s0, s1, cnt, wf1, bf1, wf2, bf2, out_ref):
    denom = jnp.maximum(cnt[...], 1.0)
    p0 = s0[...] / denom
    p1 = s1[...] / denom
    h = jnp.maximum(
        jnp.dot(p0, wf1[...][0:HH, :], preferred_element_type=F32)
        + jnp.dot(p1, wf1[...][HH:H, :], preferred_element_type=F32)
        + bf1[...], 0.0)
    logits = jnp.dot(h, wf2[...], preferred_element_type=F32) + bf2[...]
    m = jnp.max(logits, axis=1, keepdims=True)
    e = jnp.exp(logits - m)
    lse = jnp.log(jnp.sum(e, axis=1, keepdims=True)) + m
    out_ref[...] = logits - lse


def _head_call(s0, s1, cnt, wf1, bf1, wf2, bf2):
    return pl.pallas_call(
        _head_body,
        out_shape=jax.ShapeDtypeStruct((NG, 10), F32),
    )(s0, s1, cnt, wf1, bf1, wf2, bf2)


def kernel(x, edge_index, batch, W1, b1, W2, b2, W3, b3, Wf1, bf1, Wf2, bf2):
    src = edge_index[0]
    dst = edge_index[1]
    xp = jnp.pad(x, ((0, NPAD - N), (0, 0)))
    batchp = jnp.pad(batch, (0, NPAD - N), constant_values=-1).reshape(NPAD, 1)

    dega, degb = _deg_call()(dst)
    dinv, gx = _prep_call(xp, dega.reshape(NPAD, 1), degb.reshape(NPAD, 1))

    z8 = jnp.zeros((NPAD, 8), F32)
    t0a, t0b = _agg8_call()(gx, src, dst, z8)
    g1a, g1b = _l1_call(t0a, t0b, gx, dinv, W1, b1.reshape(1, H), W2)

    t1a, t1b = _agg32_call()(g1a, g1b, src, dst)
    g2a, g2b = _l2_call(t1a, t1b, g1a, g1b, dinv, b2.reshape(1, H), W3)

    t2a, t2b = _agg32_call()(g2a, g2b, src, dst)
    s0, s1, cnt = _pool_call(t2a, t2b, g2a, g2b, dinv, b3.reshape(1, H),
                             batchp)

    return _head_call(s0, s1, cnt, Wf1, bf1.reshape(1, HH), Wf2,
                      bf2.reshape(1, 10))


# speedup vs baseline: 28.7359x; 1.2449x over previous
import functools

import jax
import jax.numpy as jnp
from jax import lax
from jax.experimental import pallas as pl
from jax.experimental.pallas import tpu as pltpu
from jax.experimental.pallas import tpu_sc as plsc

N = 50000
E = 800000
NG = 512
H = 64
HH = 32

NC = 2
NS = 16
SLAB = 3136
NPAD = NS * SLAB
BN = SLAB
GRID = NPAD // BN

F32 = jnp.float32

@functools.cache
def _mesh():
    return plsc.VectorSubcoreMesh(core_axis_name="c", subcore_axis_name="s",
                                  num_cores=NC, num_subcores=NS)


EC = E // (NC * NS)
DC = 1000
EPS = E // NS
AC = 400
STG = 392


def _fill_f32(ref, n, val):
    def body(i, _):
        ref[pl.ds(i * 16, 16)] = jnp.full((16,), val, F32)
        return 0
    lax.fori_loop(0, n // 16, body, 0)


def _deg_body(dst_hbm, out0, out1, dst_v, ones_v, zrow_v, acc, sem):
    c = lax.axis_index("c")
    s = lax.axis_index("s")
    _fill_f32(zrow_v, SLAB, 0.0)
    _fill_f32(ones_v, DC, 1.0)
    pltpu.sync_copy(zrow_v, acc.at[pl.ds(s * SLAB, SLAB)])
    plsc.subcore_barrier()
    wid = s * NC + c
    def chunk(k, _):
        base = wid * EC + k * DC
        pltpu.sync_copy(dst_hbm.at[pl.ds(base, DC)], dst_v)
        pltpu.async_copy(ones_v, acc.at[dst_v], sem, add=True).wait()
        return 0
    lax.fori_loop(0, EC // DC, chunk, 0)
    plsc.subcore_barrier()
    sl = pl.ds(s * SLAB, SLAB)
    pltpu.sync_copy(acc.at[sl], zrow_v)
    @pl.when(c == 0)
    def _():
        pltpu.sync_copy(zrow_v, out0.at[sl])
    @pl.when(c == 1)
    def _():
        pltpu.sync_copy(zrow_v, out1.at[sl])


@functools.cache
def _deg_call():
    return pl.kernel(
    _deg_body,
    out_type=(jax.ShapeDtypeStruct((NPAD,), F32),
              jax.ShapeDtypeStruct((NPAD,), F32)),
    mesh=_mesh(),
    scratch_types=[
        pltpu.VMEM((DC,), jnp.int32),
        pltpu.VMEM((DC,), F32),
        pltpu.VMEM((SLAB,), F32),
        pltpu.VMEM_SHARED((NPAD,), F32),
        pltpu.SemaphoreType.DMA,
    ],
    compiler_params=pltpu.CompilerParams(use_tc_tiling_on_sc=False),
    )


def _agg8_body(tab_hbm, src_hbm, dst_hbm, z_hbm, out0, out1,
               src_v, dst_v, rows_v, slab_v, acc, sem):
    c = lax.axis_index("c")
    s = lax.axis_index("s")
    for j in range(SLAB // 784):
        zsl = pl.ds(s * SLAB + j * 784, 784)
        pltpu.sync_copy(z_hbm.at[zsl], slab_v)
        pltpu.sync_copy(slab_v, acc.at[zsl])
    plsc.subcore_barrier()
    wid = s * NC + c
    def chunk(k, _):
        base = wid * EC + k * DC
        pltpu.sync_copy(src_hbm.at[pl.ds(base, DC)], src_v)
        pltpu.sync_copy(dst_hbm.at[pl.ds(base, DC)], dst_v)
        pltpu.async_copy(tab_hbm.at[src_v], rows_v, sem).wait()
        pltpu.async_copy(rows_v, acc.at[dst_v], sem, add=True).wait()
        return 0
    lax.fori_loop(0, EC // DC, chunk, 0)
    plsc.subcore_barrier()
    for j in range(SLAB // 784):
        sl = pl.ds(s * SLAB + j * 784, 784)
        pltpu.sync_copy(acc.at[sl], slab_v)
        @pl.when(c == 0)
        def _():
            pltpu.sync_copy(slab_v, out0.at[sl])
        @pl.when(c == 1)
        def _():
            pltpu.sync_copy(slab_v, out1.at[sl])


@functools.cache
def _agg8_call():
    return pl.kernel(
    _agg8_body,
    out_type=(jax.ShapeDtypeStruct((NPAD, 8), F32),
              jax.ShapeDtypeStruct((NPAD, 8), F32)),
    mesh=_mesh(),
    scratch_types=[
        pltpu.VMEM((DC,), jnp.int32),
        pltpu.VMEM((DC,), jnp.int32),
        pltpu.VMEM((DC, 8), F32),
        pltpu.VMEM((784, 8), F32),
        pltpu.VMEM_SHARED((NPAD, 8), F32),
        pltpu.SemaphoreType.DMA,
    ],
    compiler_params=pltpu.CompilerParams(use_tc_tiling_on_sc=False),
    )


def _fill_rows32(ref, nrows):
    def body(i, _):
        ref[i, pl.ds(0, 16)] = jnp.zeros((16,), F32)
        ref[i, pl.ds(16, 16)] = jnp.zeros((16,), F32)
        return 0
    lax.fori_loop(0, nrows, body, 0)


def _agg32_body(taba_hbm, tabb_hbm, srcd_hbm, dstd_hbm, outa, outb,
                src_a, src_b, dst_a, dst_b, rows_a, rows_b, acc,
                sem_a, sem_b):
    c = lax.axis_index("c")
    s = lax.axis_index("s")
    stg = rows_a.at[pl.ds(0, STG)]
    nch = EPS // AC

    def half(tab_hbm, out):
        _fill_rows32(rows_a, STG)
        for j in range(SLAB // STG):
            pltpu.sync_copy(stg, acc.at[pl.ds(s * SLAB + j * STG, STG)])
        plsc.subcore_barrier()
        ebase = s * EPS
        pltpu.sync_copy(srcd_hbm.at[pl.ds(ebase, AC)], src_a)
        pltpu.async_copy(tab_hbm.at[src_a], rows_a, sem_a)

        def pair(m, _):
            b0 = ebase + (2 * m) * AC
            pltpu.sync_copy(srcd_hbm.at[pl.ds(b0 + AC, AC)], src_b)
            pltpu.async_copy(tab_hbm.at[src_b], rows_b, sem_b)
            pltpu.sync_copy(dstd_hbm.at[pl.ds(b0, AC)], dst_a)
            pltpu.make_async_copy(tab_hbm.at[src_a], rows_a, sem_a).wait()
            pltpu.sync_copy(rows_a, acc.at[dst_a], add=True)
            pltpu.sync_copy(srcd_hbm.at[pl.ds(b0 + 2 * AC, AC)], src_a)
            pltpu.async_copy(tab_hbm.at[src_a], rows_a, sem_a)
            pltpu.sync_copy(dstd_hbm.at[pl.ds(b0 + AC, AC)], dst_b)
            pltpu.make_async_copy(tab_hbm.at[src_b], rows_b, sem_b).wait()
            pltpu.sync_copy(rows_b, acc.at[dst_b], add=True)
            return 0

        lax.fori_loop(0, (nch - 1) // 2, pair, 0)
        bl = ebase + (nch - 1) * AC
        pltpu.sync_copy(dstd_hbm.at[pl.ds(bl, AC)], dst_a)
        pltpu.make_async_copy(tab_hbm.at[src_a], rows_a, sem_a).wait()
        pltpu.sync_copy(rows_a, acc.at[dst_a], add=True)
        plsc.subcore_barrier()
        for j in range(SLAB // STG):
            sl = pl.ds(s * SLAB + j * STG, STG)
            pltpu.sync_copy(acc.at[sl], stg)
            pltpu.sync_copy(stg, out.at[sl])

    @pl.when(c == 0)
    def _():
        half(taba_hbm, outa)
    @pl.when(c == 1)
    def _():
        half(tabb_hbm, outb)


@functools.cache
def _agg32_call():
    return pl.kernel(
    _agg32_body,
    out_type=(jax.ShapeDtypeStruct((NPAD, HH), F32),
              jax.ShapeDtypeStruct((NPAD, HH), F32)),
    mesh=_mesh(),
    scratch_types=[
        pltpu.VMEM((AC,), jnp.int32),
        pltpu.VMEM((AC,), jnp.int32),
        pltpu.VMEM((AC,), jnp.int32),
        pltpu.VMEM((AC,), jnp.int32),
        pltpu.VMEM((AC, HH), F32),
        pltpu.VMEM((AC, HH), F32),
        pltpu.VMEM_SHARED((NPAD, HH), F32),
        pltpu.SemaphoreType.DMA,
        pltpu.SemaphoreType.DMA,
    ],
    compiler_params=pltpu.CompilerParams(use_tc_tiling_on_sc=False),
    )


def _row_spec(w):
    return pl.BlockSpec((BN, w), lambda i: (i, 0))


def _full_spec(a, b):
    return pl.BlockSpec((a, b), lambda i: (0, 0))


def _prep_body(x_ref, da_ref, db_ref, dinv_ref, gx_ref):
    deg = da_ref[...] + db_ref[...] + 1.0
    dinv = lax.rsqrt(deg)
    dinv_ref[...] = dinv
    xs = x_ref[...] * dinv
    gx_ref[...] = jnp.concatenate([xs, jnp.zeros((BN, 6), F32)], axis=1)


def _prep_call(xp, da, db):
    return pl.pallas_call(
        _prep_body,
        grid=(GRID,),
        in_specs=[_row_spec(2), _row_spec(1), _row_spec(1)],
        out_specs=(_row_spec(1), _row_spec(8)),
        out_shape=(jax.ShapeDtypeStruct((NPAD, 1), F32),
                   jax.ShapeDtypeStruct((NPAD, 8), F32)),
    )(xp, da, db)


def _l1_body(t0a, t0b, gx, dinv, w1, b1, w2, ga_ref, gb_ref):
    q = t0a[...] + t0b[...] + gx[...]
    xin = q[:, 0:2] * dinv[...]
    y1 = jnp.maximum(
        jnp.dot(xin, w1[...], preferred_element_type=F32) + b1[...], 0.0)
    h2 = jnp.dot(y1, w2[...], preferred_element_type=F32)
    g1 = h2 * dinv[...]
    ga_ref[...] = g1[:, 0:HH]
    gb_ref[...] = g1[:, HH:H]


def _l1_call(t0a, t0b, gx, dinv, w1, b1, w2):
    return pl.pallas_call(
        _l1_body,
        grid=(GRID,),
        in_specs=[_row_spec(8), _row_spec(8), _row_spec(8), _row_spec(1),
                  _full_spec(2, H), _full_spec(1, H), _full_spec(H, H)],
        out_specs=(_row_spec(HH), _row_spec(HH)),
        out_shape=(jax.ShapeDtypeStruct((NPAD, HH), F32),
                   jax.ShapeDtypeStruct((NPAD, HH), F32)),
    )(t0a, t0b, gx, dinv, w1, b1, w2)


def _l2_body(ta, tb, ga, gb, dinv, b2, w3, oa_ref, ob_ref):
    d = dinv[...]
    u0 = jnp.maximum((ta[...] + ga[...]) * d + b2[...][:, 0:HH], 0.0)
    u1 = jnp.maximum((tb[...] + gb[...]) * d + b2[...][:, HH:H], 0.0)
    h3 = (jnp.dot(u0, w3[...][0:HH, :], preferred_element_type=F32)
          + jnp.dot(u1, w3[...][HH:H, :], preferred_element_type=F32))
    g2 = h3 * d
    oa_ref[...] = g2[:, 0:HH]
    ob_ref[...] = g2[:, HH:H]


def _l2_call(ta, tb, ga, gb, dinv, b2, w3):
    return pl.pallas_call(
        _l2_body,
        grid=(GRID,),
        in_specs=[_row_spec(HH)] * 4 + [_row_spec(1),
                  _full_spec(1, H), _full_spec(H, H)],
        out_specs=(_row_spec(HH), _row_spec(HH)),
        out_shape=(jax.ShapeDtypeStruct((NPAD, HH), F32),
                   jax.ShapeDtypeStruct((NPAD, HH), F32)),
    )(ta, tb, ga, gb, dinv, b2, w3)


def _pool_body(ta, tb, ga, gb, dinv, b3, batch, s0_ref, s1_ref, cnt_ref):
    i = pl.program_id(0)
    d = dinv[...]
    v0 = jnp.maximum((ta[...] + ga[...]) * d + b3[...][:, 0:HH], 0.0)
    v1 = jnp.maximum((tb[...] + gb[...]) * d + b3[...][:, HH:H], 0.0)
    ids = batch[...]
    seg = lax.broadcasted_iota(jnp.int32, (BN, NG), 1)
    p = (ids == seg).astype(F32)
    dn = (((0,), (0,)), ((), ()))
    s0 = lax.dot_general(p, v0, dn, preferred_element_type=F32)
    s1 = lax.dot_general(p, v1, dn, preferred_element_type=F32)
    cnt = lax.dot_general(p, jnp.ones((BN, 1), F32), dn,
                          preferred_element_type=F32)

    @pl.when(i == 0)
    def _():
        s0_ref[...] = jnp.zeros_like(s0_ref)
        s1_ref[...] = jnp.zeros_like(s1_ref)
        cnt_ref[...] = jnp.zeros_like(cnt_ref)

    s0_ref[...] += s0
    s1_ref[...] += s1
    cnt_ref[...] += cnt


def _pool_call(ta, tb, ga, gb, dinv, b3, batchp):
    return pl.pallas_call(
        _pool_body,
        grid=(GRID,),
        in_specs=[_row_spec(HH)] * 4 + [_row_spec(1), _full_spec(1, H),
                  pl.BlockSpec((BN, 1), lambda i: (i, 0))],
        out_specs=(_full_spec(NG, HH), _full_spec(NG, HH), _full_spec(NG, 1)),
        out_shape=(jax.ShapeDtypeStruct((NG, HH), F32),
                   jax.ShapeDtypeStruct((NG, HH), F32),
                   jax.ShapeDtypeStruct((NG, 1), F32)),
    )(ta, tb, ga, gb, dinv, b3, batchp)


def _head_body(s0, s1, cnt, wf1, bf1, wf2, bf2, out_ref):
    denom = jnp.maximum(cnt[...], 1.0)
    p0 = s0[...] / denom
    p1 = s1[...] / denom
    h = jnp.maximum(
        jnp.dot(p0, wf1[...][0:HH, :], preferred_element_type=F32)
        + jnp.dot(p1, wf1[...][HH:H, :], preferred_element_type=F32)
        + bf1[...], 0.0)
    logits = jnp.dot(h, wf2[...], preferred_element_type=F32) + bf2[...]
    m = jnp.max(logits, axis=1, keepdims=True)
    e = jnp.exp(logits - m)
    lse = jnp.log(jnp.sum(e, axis=1, keepdims=True)) + m
    out_ref[...] = logits - lse


def _head_call(s0, s1, cnt, wf1, bf1, wf2, bf2):
    return pl.pallas_call(
        _head_body,
        out_shape=jax.ShapeDtypeStruct((NG, 10), F32),
    )(s0, s1, cnt, wf1, bf1, wf2, bf2)


def kernel(x, edge_index, batch, W1, b1, W2, b2, W3, b3, Wf1, bf1, Wf2, bf2):
    src = edge_index[0]
    dst = edge_index[1]
    xp = jnp.pad(x, ((0, NPAD - N), (0, 0)))
    batchp = jnp.pad(batch, (0, NPAD - N), constant_values=-1).reshape(NPAD, 1)

    dega, degb = _deg_call()(dst)
    dinv, gx = _prep_call(xp, dega.reshape(NPAD, 1), degb.reshape(NPAD, 1))

    z8 = jnp.zeros((NPAD, 8), F32)
    t0a, t0b = _agg8_call()(gx, src, dst, z8)
    g1a, g1b = _l1_call(t0a, t0b, gx, dinv, W1, b1.reshape(1, H), W2)

    t1a, t1b = _agg32_call()(g1a, g1b, src, dst)
    g2a, g2b = _l2_call(t1a, t1b, g1a, g1b, dinv, b2.reshape(1, H), W3)

    t2a, t2b = _agg32_call()(g2a, g2b, src, dst)
    s0, s1, cnt = _pool_call(t2a, t2b, g2a, g2b, dinv, b3.reshape(1, H),
                             batchp)

    return _head_call(s0, s1, cnt, Wf1, bf1.reshape(1, HH), Wf2,
                      bf2.reshape(1, 10))


# speedup vs baseline: 35.4460x; 1.2335x over previous
import functools

import jax
import jax.numpy as jnp
from jax import lax
from jax.experimental import pallas as pl
from jax.experimental.pallas import tpu as pltpu
from jax.experimental.pallas import tpu_sc as plsc

N = 50000
E = 800000
NG = 512
H = 64
HH = 32

NC = 2
NS = 16
SLAB = 3136
NPAD = NS * SLAB
BN = SLAB
GRID = NPAD // BN
PK = NPAD * HH // 128
PB = PK // GRID

F32 = jnp.float32


@functools.cache
def _mesh():
    return plsc.VectorSubcoreMesh(core_axis_name="c", subcore_axis_name="s",
                                  num_cores=NC, num_subcores=NS)


EC = E // (NC * NS)
DC = 1000
AC0 = 200
STG0 = 196
EPS = E // NS
AC = 400
STG = 392


def _fill_f32(ref, n, val):
    def body(i, _):
        ref[pl.ds(i * 16, 16)] = jnp.full((16,), val, F32)
        return 0
    lax.fori_loop(0, n // 16, body, 0)


def _fill_rows32(ref, nrows):
    def body(i, _):
        ref[i, pl.ds(0, 16)] = jnp.zeros((16,), F32)
        ref[i, pl.ds(16, 16)] = jnp.zeros((16,), F32)
        return 0
    lax.fori_loop(0, nrows, body, 0)


def _deg_body(dst_hbm, out0, out1, dst_v, ones_v, zrow_v, acc, sem):
    c = lax.axis_index("c")
    s = lax.axis_index("s")
    _fill_f32(zrow_v, SLAB, 0.0)
    _fill_f32(ones_v, DC, 1.0)
    pltpu.sync_copy(zrow_v, acc.at[pl.ds(s * SLAB, SLAB)])
    plsc.subcore_barrier()
    wid = s * NC + c
    def chunk(k, _):
        base = wid * EC + k * DC
        pltpu.sync_copy(dst_hbm.at[pl.ds(base, DC)], dst_v)
        pltpu.async_copy(ones_v, acc.at[dst_v], sem, add=True).wait()
        return 0
    lax.fori_loop(0, EC // DC, chunk, 0)
    plsc.subcore_barrier()
    sl = pl.ds(s * SLAB, SLAB)
    pltpu.sync_copy(acc.at[sl], zrow_v)
    @pl.when(c == 0)
    def _():
        pltpu.sync_copy(zrow_v, out0.at[sl])
    @pl.when(c == 1)
    def _():
        pltpu.sync_copy(zrow_v, out1.at[sl])


@functools.cache
def _deg_call():
    return pl.kernel(
        _deg_body,
        out_type=(jax.ShapeDtypeStruct((NPAD,), F32),
                  jax.ShapeDtypeStruct((NPAD,), F32)),
        mesh=_mesh(),
        scratch_types=[
            pltpu.VMEM((DC,), jnp.int32),
            pltpu.VMEM((DC,), F32),
            pltpu.VMEM((SLAB,), F32),
            pltpu.VMEM_SHARED((NPAD,), F32),
            pltpu.SemaphoreType.DMA,
        ],
        compiler_params=pltpu.CompilerParams(use_tc_tiling_on_sc=False),
    )


def _agg_pipeline(tab_hbm, src_hbm, dst_hbm, acc, s, ebase, nch, ac,
                  src_a, src_b, dst_a, dst_b, rows_a, rows_b, sem_a, sem_b):
    pltpu.sync_copy(src_hbm.at[pl.ds(ebase, ac)], src_a)
    pltpu.async_copy(tab_hbm.at[src_a], rows_a, sem_a)

    def pair(m, _):
        b0 = ebase + (2 * m) * ac
        pltpu.sync_copy(src_hbm.at[pl.ds(b0 + ac, ac)], src_b)
        pltpu.async_copy(tab_hbm.at[src_b], rows_b, sem_b)
        pltpu.sync_copy(dst_hbm.at[pl.ds(b0, ac)], dst_a)
        pltpu.make_async_copy(tab_hbm.at[src_a], rows_a, sem_a).wait()
        pltpu.sync_copy(rows_a, acc.at[dst_a], add=True)
        pltpu.sync_copy(src_hbm.at[pl.ds(b0 + 2 * ac, ac)], src_a)
        pltpu.async_copy(tab_hbm.at[src_a], rows_a, sem_a)
        pltpu.sync_copy(dst_hbm.at[pl.ds(b0 + ac, ac)], dst_b)
        pltpu.make_async_copy(tab_hbm.at[src_b], rows_b, sem_b).wait()
        pltpu.sync_copy(rows_b, acc.at[dst_b], add=True)
        return 0

    lax.fori_loop(0, (nch - 1) // 2, pair, 0)
    bl = ebase + (nch - 1) * ac
    pltpu.sync_copy(dst_hbm.at[pl.ds(bl, ac)], dst_a)
    pltpu.make_async_copy(tab_hbm.at[src_a], rows_a, sem_a).wait()
    pltpu.sync_copy(rows_a, acc.at[dst_a], add=True)


def _aggl0_body(tab_hbm, srcd_hbm, dstd_hbm, out0, out1,
                src_a, src_b, dst_a, dst_b, rows_a, rows_b, acc,
                sem_a, sem_b):
    c = lax.axis_index("c")
    s = lax.axis_index("s")
    stg = rows_a.at[pl.ds(0, STG0)]
    _fill_rows32(rows_a, STG0)
    for j in range(SLAB // STG0):
        pltpu.sync_copy(stg, acc.at[pl.ds(s * SLAB + j * STG0, STG0)])
    plsc.subcore_barrier()
    wid = s * NC + c
    _agg_pipeline(tab_hbm, srcd_hbm, dstd_hbm, acc, s, wid * EC,
                  EC // AC0, AC0,
                  src_a, src_b, dst_a, dst_b, rows_a, rows_b, sem_a, sem_b)
    plsc.subcore_barrier()
    for j in range(SLAB // STG0):
        sl = pl.ds(s * SLAB + j * STG0, STG0)
        pltpu.sync_copy(acc.at[sl], stg)
        @pl.when(c == 0)
        def _():
            pltpu.sync_copy(stg, out0.at[sl])
        @pl.when(c == 1)
        def _():
            pltpu.sync_copy(stg, out1.at[sl])


@functools.cache
def _aggl0_call():
    return pl.kernel(
        _aggl0_body,
        out_type=(jax.ShapeDtypeStruct((NPAD, HH), F32),
                  jax.ShapeDtypeStruct((NPAD, HH), F32)),
        mesh=_mesh(),
        scratch_types=[
            pltpu.VMEM((AC0,), jnp.int32),
            pltpu.VMEM((AC0,), jnp.int32),
            pltpu.VMEM((AC0,), jnp.int32),
            pltpu.VMEM((AC0,), jnp.int32),
            pltpu.VMEM((AC0, HH), F32),
            pltpu.VMEM((AC0, HH), F32),
            pltpu.VMEM_SHARED((NPAD, HH), F32),
            pltpu.SemaphoreType.DMA,
            pltpu.SemaphoreType.DMA,
        ],
        compiler_params=pltpu.CompilerParams(use_tc_tiling_on_sc=False),
    )


def _agg32_body(taba_hbm, tabb_hbm, srcd_hbm, dstd_hbm, outa, outb,
                src_a, src_b, dst_a, dst_b, rows_a, rows_b, acc,
                sem_a, sem_b):
    c = lax.axis_index("c")
    s = lax.axis_index("s")
    stg = rows_a.at[pl.ds(0, STG)]

    def half(tab_hbm, out):
        _fill_rows32(rows_a, STG)
        for j in range(SLAB // STG):
            pltpu.sync_copy(stg, acc.at[pl.ds(s * SLAB + j * STG, STG)])
        plsc.subcore_barrier()
        _agg_pipeline(tab_hbm, srcd_hbm, dstd_hbm, acc, s, s * EPS,
                      EPS // AC, AC,
                      src_a, src_b, dst_a, dst_b, rows_a, rows_b,
                      sem_a, sem_b)
        plsc.subcore_barrier()
        for j in range(SLAB // STG):
            sl = pl.ds(s * SLAB + j * STG, STG)
            pltpu.sync_copy(acc.at[sl], stg)
            pltpu.sync_copy(stg, out.at[sl])

    @pl.when(c == 0)
    def _():
        half(taba_hbm, outa)
    @pl.when(c == 1)
    def _():
        half(tabb_hbm, outb)


@functools.cache
def _agg32_call():
    return pl.kernel(
        _agg32_body,
        out_type=(jax.ShapeDtypeStruct((NPAD, HH), F32),
                  jax.ShapeDtypeStruct((NPAD, HH), F32)),
        mesh=_mesh(),
        scratch_types=[
            pltpu.VMEM((AC,), jnp.int32),
            pltpu.VMEM((AC,), jnp.int32),
            pltpu.VMEM((AC,), jnp.int32),
            pltpu.VMEM((AC,), jnp.int32),
            pltpu.VMEM((AC, HH), F32),
            pltpu.VMEM((AC, HH), F32),
            pltpu.VMEM_SHARED((NPAD, HH), F32),
            pltpu.SemaphoreType.DMA,
            pltpu.SemaphoreType.DMA,
        ],
        compiler_params=pltpu.CompilerParams(use_tc_tiling_on_sc=False),
    )


def _prow(w=128):
    return pl.BlockSpec((PB, w), lambda i: (i, 0))


def _full_spec(a, b):
    return pl.BlockSpec((a, b), lambda i: (0, 0))


def _prep_body(xw_ref, da_ref, db_ref, dinv_ref, gx_ref):
    r = lax.rsqrt(da_ref[...] + db_ref[...] + 1.0)
    dv = jnp.concatenate(
        [jnp.broadcast_to(r[:, j:j + 1], (PB, HH)) for j in range(4)], axis=1)
    dinv_ref[...] = dv
    gx_ref[...] = xw_ref[...] * dv


def _prep_call(xwp, da, db):
    return pl.pallas_call(
        _prep_body,
        grid=(GRID,),
        in_specs=[_prow(), _prow(4), _prow(4)],
        out_specs=(_prow(), _prow()),
        out_shape=(jax.ShapeDtypeStruct((PK, 128), F32),
                   jax.ShapeDtypeStruct((PK, 128), F32)),
    )(xwp, da, db)


def _mm(a, b):
    return jnp.dot(a, b, preferred_element_type=F32)


def _l1_body(t0a, t0b, gx, dinv, w1a, w1b, b1a, b1b,
             w2aa, w2ab, w2ba, w2bb, ga_ref, gb_ref):
    d = dinv[...]
    q = (t0a[...] + t0b[...] + gx[...]) * d
    y1a = jnp.maximum(_mm(q, w1a[...]) + b1a[...], 0.0)
    y1b = jnp.maximum(_mm(q, w1b[...]) + b1b[...], 0.0)
    ga_ref[...] = (_mm(y1a, w2aa[...]) + _mm(y1b, w2ba[...])) * d
    gb_ref[...] = (_mm(y1a, w2ab[...]) + _mm(y1b, w2bb[...])) * d


def _l1_call(t0a, t0b, gx, dinv, w1a, w1b, b1a, b1b, w2aa, w2ab, w2ba, w2bb):
    return pl.pallas_call(
        _l1_body,
        grid=(GRID,),
        in_specs=[_prow()] * 4 + [_full_spec(128, 128)] * 2
        + [_full_spec(1, 128)] * 2 + [_full_spec(128, 128)] * 4,
        out_specs=(_prow(), _prow()),
        out_shape=(jax.ShapeDtypeStruct((PK, 128), F32),
                   jax.ShapeDtypeStruct((PK, 128), F32)),
    )(t0a, t0b, gx, dinv, w1a, w1b, b1a, b1b, w2aa, w2ab, w2ba, w2bb)


def _l2_body(ta, tb, ga, gb, dinv, b2a, b2b,
             w3aa, w3ab, w3ba, w3bb, oa_ref, ob_ref):
    d = dinv[...]
    u0 = jnp.maximum((ta[...] + ga[...]) * d + b2a[...], 0.0)
    u1 = jnp.maximum((tb[...] + gb[...]) * d + b2b[...], 0.0)
    oa_ref[...] = (_mm(u0, w3aa[...]) + _mm(u1, w3ba[...])) * d
    ob_ref[...] = (_mm(u0, w3ab[...]) + _mm(u1, w3bb[...])) * d


def _l2_call(ta, tb, ga, gb, dinv, b2a, b2b, w3aa, w3ab, w3ba, w3bb):
    return pl.pallas_call(
        _l2_body,
        grid=(GRID,),
        in_specs=[_prow()] * 5 + [_full_spec(1, 128)] * 2
        + [_full_spec(128, 128)] * 4,
        out_specs=(_prow(), _prow()),
        out_shape=(jax.ShapeDtypeStruct((PK, 128), F32),
                   jax.ShapeDtypeStruct((PK, 128), F32)),
    )(ta, tb, ga, gb, dinv, b2a, b2b, w3aa, w3ab, w3ba, w3bb)


def _pool_body(ta, tb, ga, gb, dinv, b3a, b3b, batch,
               s0_ref, s1_ref, cnt_ref):
    i = pl.program_id(0)
    d = dinv[...]
    v0 = jnp.maximum((ta[...] + ga[...]) * d + b3a[...], 0.0)
    v1 = jnp.maximum((tb[...] + gb[...]) * d + b3b[...], 0.0)
    ids = batch[...]
    seg = lax.broadcasted_iota(jnp.int32, (PB, NG), 1)
    dn = (((0,), (0,)), ((), ()))
    ones = jnp.ones((PB, 1), F32)
    s0 = jnp.zeros((NG, HH), F32)
    s1 = jnp.zeros((NG, HH), F32)
    cnt = jnp.zeros((NG, 1), F32)
    for j in range(4):
        p = (ids[:, j:j + 1] == seg).astype(F32)
        s0 = s0 + lax.dot_general(p, v0[:, HH * j:HH * (j + 1)], dn,
                                  preferred_element_type=F32)
        s1 = s1 + lax.dot_general(p, v1[:, HH * j:HH * (j + 1)], dn,
                                  preferred_element_type=F32)
        cnt = cnt + lax.dot_general(p, ones, dn,
                                    preferred_element_type=F32)

    @pl.when(i == 0)
    def _():
        s0_ref[...] = jnp.zeros_like(s0_ref)
        s1_ref[...] = jnp.zeros_like(s1_ref)
        cnt_ref[...] = jnp.zeros_like(cnt_ref)

    s0_ref[...] += s0
    s1_ref[...] += s1
    cnt_ref[...] += cnt


def _pool_call(ta, tb, ga, gb, dinv, b3a, b3b, batchp):
    return pl.pallas_call(
        _pool_body,
        grid=(GRID,),
        in_specs=[_prow()] * 5 + [_full_spec(1, 128)] * 2
        + [pl.BlockSpec((PB, 4), lambda i: (i, 0))],
        out_specs=(_full_spec(NG, HH), _full_spec(NG, HH), _full_spec(NG, 1)),
        out_shape=(jax.ShapeDtypeStruct((NG, HH), F32),
                   jax.ShapeDtypeStruct((NG, HH), F32),
                   jax.ShapeDtypeStruct((NG, 1), F32)),
    )(ta, tb, ga, gb, dinv, b3a, b3b, batchp)


def _head_body(s0, s1, cnt, wf1, bf1, wf2, bf2, out_ref):
    denom = jnp.maximum(cnt[...], 1.0)
    p0 = s0[...] / denom
    p1 = s1[...] / denom
    h = jnp.maximum(
        _mm(p0, wf1[...][0:HH, :]) + _mm(p1, wf1[...][HH:H, :]) + bf1[...],
        0.0)
    logits = _mm(h, wf2[...]) + bf2[...]
    m = jnp.max(logits, axis=1, keepdims=True)
    e = jnp.exp(logits - m)
    lse = jnp.log(jnp.sum(e, axis=1, keepdims=True)) + m
    out_ref[...] = logits - lse


def _head_call(s0, s1, cnt, wf1, bf1, wf2, bf2):
    return pl.pallas_call(
        _head_body,
        out_shape=jax.ShapeDtypeStruct((NG, 10), F32),
    )(s0, s1, cnt, wf1, bf1, wf2, bf2)


def _bd4(w):
    return jnp.kron(jnp.eye(4, dtype=F32), w)


def _tile4(b):
    return jnp.tile(b, 4).reshape(1, 128)


def kernel(x, edge_index, batch, W1, b1, W2, b2, W3, b3, Wf1, bf1, Wf2, bf2):
    src = edge_index[0]
    dst = edge_index[1]
    xw = jnp.pad(x, ((0, NPAD - N), (0, HH - x.shape[1])))
    xwp = xw.reshape(PK, 128)
    batchp = jnp.pad(batch, (0, NPAD - N), constant_values=-1).reshape(PK, 4)

    w1pad = jnp.zeros((HH, H), F32).at[0:2, :].set(W1)
    w1a, w1b = _bd4(w1pad[:, :HH]), _bd4(w1pad[:, HH:])
    b1a, b1b = _tile4(b1[:HH]), _tile4(b1[HH:])
    w2aa, w2ab = _bd4(W2[:HH, :HH]), _bd4(W2[:HH, HH:])
    w2ba, w2bb = _bd4(W2[HH:, :HH]), _bd4(W2[HH:, HH:])
    b2a, b2b = _tile4(b2[:HH]), _tile4(b2[HH:])
    w3aa, w3ab = _bd4(W3[:HH, :HH]), _bd4(W3[:HH, HH:])
    w3ba, w3bb = _bd4(W3[HH:, :HH]), _bd4(W3[HH:, HH:])
    b3a, b3b = _tile4(b3[:HH]), _tile4(b3[HH:])

    dega, degb = _deg_call()(dst)
    dinvp, gxp = _prep_call(xwp, dega.reshape(PK, 4), degb.reshape(PK, 4))

    t0a, t0b = _aggl0_call()(gxp.reshape(NPAD, HH), src, dst)
    g1a, g1b = _l1_call(t0a.reshape(PK, 128), t0b.reshape(PK, 128),
                        gxp, dinvp, w1a, w1b, b1a, b1b,
                        w2aa, w2ab, w2ba, w2bb)

    t1a, t1b = _agg32_call()(g1a.reshape(NPAD, HH), g1b.reshape(NPAD, HH),
                             src, dst)
    g2a, g2b = _l2_call(t1a.reshape(PK, 128), t1b.reshape(PK, 128),
                        g1a, g1b, dinvp, b2a, b2b,
                        w3aa, w3ab, w3ba, w3bb)

    t2a, t2b = _agg32_call()(g2a.reshape(NPAD, HH), g2b.reshape(NPAD, HH),
                             src, dst)
    s0, s1, cnt = _pool_call(t2a.reshape(PK, 128), t2b.reshape(PK, 128),
                             g2a, g2b, dinvp, b3a, b3b, batchp)

    return _head_call(s0, s1, cnt, Wf1, bf1.reshape(1, HH), Wf2,
                      bf2.reshape(1, 10))
